# Initial kernel scaffold; baseline (speedup 1.0000x reference)
import jax
import jax.numpy as jnp
from jax.experimental import pallas as pl


def kernel(s, nbrs_s, r, candidates, nbrs_candidates, labels, entities_emb, relations_emb):
    raise NotImplementedError("write your pallas kernel here")


# speedup vs baseline: 10.6411x; 10.6411x over previous
import functools

import jax
import jax.numpy as jnp
from jax import lax
from jax.experimental import pallas as pl
from jax.experimental.pallas import tpu as pltpu
from jax.experimental.pallas import tpu_sc as plsc

L = 16
D = 64
CP = 208
CA, CB = 128, 80
NG = CP // L


def _build_sc_call(B, V, RV):
  mesh = plsc.VectorSubcoreMesh(core_axis_name="c", subcore_axis_name="s")
  nc, ns = mesh.num_cores, mesh.num_subcores
  nw = nc * ns
  ipw = B // nw

  @functools.partial(
      pl.kernel,
      mesh=mesh,
      out_type=jax.ShapeDtypeStruct((B, CP), jnp.float32),
      compiler_params=pltpu.CompilerParams(
          needs_layout_passes=False, use_tc_tiling_on_sc=False),
      scratch_types=[
          pltpu.VMEM((ipw,), jnp.int32),
          pltpu.VMEM((ipw,), jnp.int32),
          pltpu.VMEM((ipw, CA), jnp.int32),
          pltpu.VMEM((ipw, CB), jnp.int32),
          pltpu.VMEM((ipw, D), jnp.float32),
          pltpu.VMEM((ipw, D), jnp.float32),
          pltpu.VMEM((CP, D), jnp.float32),
          pltpu.VMEM((CP, D), jnp.float32),
          pltpu.VMEM((L, L), jnp.float32),
          pltpu.VMEM((ipw, CP), jnp.float32),
          pltpu.SemaphoreType.DMA,
          pltpu.SemaphoreType.DMA,
          pltpu.SemaphoreType.DMA,
          pltpu.SemaphoreType.DMA,
          pltpu.SemaphoreType.DMA,
      ],
  )
  def sc_call(ent_hbm, rel_hbm, s_hbm, r_hbm, ca_hbm, cb_hbm, out_hbm,
              sidx_v, ridx_v, ca_v, cb_v, q_v, rrow_v, buf0, buf1, scr_v,
              sco_v, semq, sa0, sb0, sa1, sb1):
    wid = lax.axis_index("s") * nc + lax.axis_index("c")
    base = wid * ipw

    pltpu.sync_copy(s_hbm.at[pl.ds(base, ipw)], sidx_v)
    pltpu.sync_copy(r_hbm.at[pl.ds(base, ipw)], ridx_v)
    pltpu.sync_copy(ca_hbm.at[pl.ds(base, ipw)], ca_v)
    pltpu.sync_copy(cb_hbm.at[pl.ds(base, ipw)], cb_v)

    pltpu.async_copy(ent_hbm.at[sidx_v], q_v, semq).wait()
    pltpu.async_copy(rel_hbm.at[ridx_v], rrow_v, semq).wait()

    @pl.loop(0, ipw)
    def _(i):
      for k in range(D // L):
        sl = pl.ds(k * L, L)
        q_v[i, sl] = q_v[i, sl] * rrow_v[i, sl]

    def descs(i, buf, sa, sb):
      da = pltpu.make_async_copy(ent_hbm.at[ca_v.at[i]], buf.at[pl.ds(0, CA)], sa)
      db = pltpu.make_async_copy(ent_hbm.at[cb_v.at[i]], buf.at[pl.ds(CA, CB)], sb)
      return da, db

    lane = lax.iota(jnp.int32, L)

    def compute(i, buf):
      q = [q_v[i, pl.ds(k * L, L)] for k in range(D // L)]

      @pl.loop(0, NG)
      def _(g):
        row0 = g * L
        for c in range(L):
          r = row0 + c
          p = buf[r, pl.ds(0, L)] * q[0]
          for k in range(1, D // L):
            p = p + buf[r, pl.ds(k * L, L)] * q[k]
          scr_v[c, :] = p
        acc = plsc.load_gather(scr_v, [lane, jnp.full((L,), 0, jnp.int32)])
        for l in range(1, L):
          acc = acc + plsc.load_gather(scr_v, [lane, jnp.full((L,), l, jnp.int32)])
        sco_v[i, pl.ds(row0, L)] = acc

    d0a, d0b = descs(0, buf0, sa0, sb0)
    d0a.start()
    d0b.start()

    @pl.loop(0, ipw // 2)
    def _(t):
      i0 = t * 2
      i1 = i0 + 1
      n1a, n1b = descs(i1, buf1, sa1, sb1)
      n1a.start()
      n1b.start()
      w0a, w0b = descs(i0, buf0, sa0, sb0)
      w0a.wait()
      w0b.wait()
      compute(i0, buf0)

      @pl.when(i1 + 1 < ipw)
      def _():
        n0a, n0b = descs(i1 + 1, buf0, sa0, sb0)
        n0a.start()
        n0b.start()

      w1a, w1b = descs(i1, buf1, sa1, sb1)
      w1a.wait()
      w1b.wait()
      compute(i1, buf1)

    pltpu.sync_copy(sco_v, out_hbm.at[pl.ds(base, ipw)])

  return sc_call


def kernel(s, nbrs_s, r, candidates, nbrs_candidates, labels, entities_emb,
           relations_emb):
  B, C = candidates.shape
  V = entities_emb.shape[0]
  RV = relations_emb.shape[0]
  pad = jnp.full((B, CP - C), V - 1, dtype=jnp.int32)
  cpad = jnp.concatenate([candidates.astype(jnp.int32), pad], axis=1)
  ca = cpad[:, :CA]
  cb = cpad[:, CA:]
  out = _build_sc_call(B, V, RV)(
      entities_emb, relations_emb, s.astype(jnp.int32), r.astype(jnp.int32),
      ca, cb)
  return out[:, :C]


# speedup vs baseline: 19.3243x; 1.8160x over previous
import functools

import jax
import jax.numpy as jnp
from jax import lax
from jax.experimental import pallas as pl
from jax.experimental.pallas import tpu as pltpu
from jax.experimental.pallas import tpu_sc as plsc

L = 16
D = 64
C = 200
CP = 208
CA, CB = 128, 72
NG = CP // L
NK = D // L


def _take(v, idx):
  return lax.gather(
      v, idx[:, None],
      dimension_numbers=lax.GatherDimensionNumbers(
          offset_dims=(), collapsed_slice_dims=(0,), start_index_map=(0,)),
      slice_sizes=(1,),
      mode=lax.GatherScatterMode.PROMISE_IN_BOUNDS)


def _build_sc_call(B, V, RV):
  mesh = plsc.VectorSubcoreMesh(core_axis_name="c", subcore_axis_name="s")
  nc, ns = mesh.num_cores, mesh.num_subcores
  nw = nc * ns
  ipw = B // nw

  @functools.partial(
      pl.kernel,
      mesh=mesh,
      out_type=jax.ShapeDtypeStruct((B, C), jnp.float32),
      compiler_params=pltpu.CompilerParams(
          needs_layout_passes=False, use_tc_tiling_on_sc=False),
      scratch_types=[
          pltpu.VMEM((ipw,), jnp.int32),
          pltpu.VMEM((ipw,), jnp.int32),
          pltpu.VMEM((ipw, C), jnp.int32),
          pltpu.VMEM((ipw, D), jnp.float32),
          pltpu.VMEM((ipw, D), jnp.float32),
          pltpu.VMEM((CP, D), jnp.float32),
          pltpu.VMEM((CP, D), jnp.float32),
          pltpu.VMEM((ipw, CP), jnp.float32),
          pltpu.SemaphoreType.DMA,
          pltpu.SemaphoreType.DMA,
          pltpu.SemaphoreType.DMA,
          pltpu.SemaphoreType.DMA,
          pltpu.SemaphoreType.DMA,
      ],
  )
  def sc_call(ent_hbm, rel_hbm, s_hbm, r_hbm, cand_hbm, out_hbm,
              sidx_v, ridx_v, cidx_v, q_v, rrow_v, buf0, buf1,
              sco_v, semq, sa0, sb0, sa1, sb1):
    wid = lax.axis_index("s") * nc + lax.axis_index("c")
    base = wid * ipw

    pltpu.sync_copy(s_hbm.at[pl.ds(base, ipw)], sidx_v)
    pltpu.sync_copy(r_hbm.at[pl.ds(base, ipw)], ridx_v)
    pltpu.sync_copy(cand_hbm.at[pl.ds(base, ipw)], cidx_v)

    pltpu.async_copy(ent_hbm.at[sidx_v], q_v, semq).wait()
    pltpu.async_copy(rel_hbm.at[ridx_v], rrow_v, semq).wait()

    @pl.loop(0, ipw)
    def _(i):
      for k in range(NK):
        sl = pl.ds(k * L, L)
        q_v[i, sl] = q_v[i, sl] * rrow_v[i, sl]

    def descs(i, buf, sa, sb):
      da = pltpu.make_async_copy(
          ent_hbm.at[cidx_v.at[i, pl.ds(0, CA)]], buf.at[pl.ds(0, CA)], sa)
      db = pltpu.make_async_copy(
          ent_hbm.at[cidx_v.at[i, pl.ds(CA, CB)]], buf.at[pl.ds(CA, CB)], sb)
      return da, db

    lane = lax.iota(jnp.int32, L)
    xors = [lane ^ (1 << lv) for lv in range(4)]
    picks = [(lane & (1 << lv)) != 0 for lv in range(4)]

    def compute(i, buf):
      q = [q_v[i, pl.ds(k * L, L)] for k in range(NK)]

      @pl.loop(0, NG)
      def _(g):
        row0 = g * L
        ps = []
        for c in range(L):
          r = row0 + c
          p = buf[r, pl.ds(0, L)] * q[0]
          for k in range(1, NK):
            p = p + buf[r, pl.ds(k * L, L)] * q[k]
          ps.append(p)
        for lv in range(4):
          stride_summed = [v + _take(v, xors[lv]) for v in ps]
          ps = [
              jnp.where(picks[lv], stride_summed[2 * k + 1],
                        stride_summed[2 * k])
              for k in range(len(ps) // 2)
          ]
        sco_v[i, pl.ds(row0, L)] = ps[0]

    d0a, d0b = descs(0, buf0, sa0, sb0)
    d0a.start()
    d0b.start()

    @pl.loop(0, ipw // 2)
    def _(t):
      i0 = t * 2
      i1 = i0 + 1
      n1a, n1b = descs(i1, buf1, sa1, sb1)
      n1a.start()
      n1b.start()
      w0a, w0b = descs(i0, buf0, sa0, sb0)
      w0a.wait()
      w0b.wait()
      compute(i0, buf0)

      @pl.when(i1 + 1 < ipw)
      def _():
        n0a, n0b = descs(i1 + 1, buf0, sa0, sb0)
        n0a.start()
        n0b.start()

      w1a, w1b = descs(i1, buf1, sa1, sb1)
      w1a.wait()
      w1b.wait()
      compute(i1, buf1)

    pltpu.sync_copy(sco_v.at[:, pl.ds(0, C)], out_hbm.at[pl.ds(base, ipw)])

  return sc_call


def kernel(s, nbrs_s, r, candidates, nbrs_candidates, labels, entities_emb,
           relations_emb):
  B = candidates.shape[0]
  V = entities_emb.shape[0]
  RV = relations_emb.shape[0]
  return _build_sc_call(B, V, RV)(
      entities_emb, relations_emb, s.astype(jnp.int32), r.astype(jnp.int32),
      candidates.astype(jnp.int32))
